# Initial kernel scaffold; baseline (speedup 1.0000x reference)
import jax
import jax.numpy as jnp
from jax.experimental import pallas as pl


def kernel(phase, length, base, w_ps):
    raise NotImplementedError("write your pallas kernel here")


# speedup vs baseline: 5.1474x; 5.1474x over previous
import functools

import jax
import jax.numpy as jnp
from jax import lax
from jax.experimental import pallas as pl
from jax.experimental.pallas import tpu as pltpu
from jax.experimental.pallas import tpu_sc as plsc

_CYCLE = 1000
_L = 200
_D = 64
_B = 4096
_EXT = _CYCLE + _L - 1

_NC = 2
_NS = 16
_NW = _NC * _NS
_BPW = _B // _NW
_LANES = 16


@functools.partial(
    pl.kernel,
    out_type=jax.ShapeDtypeStruct((_B, _L, _D), jnp.float32),
    mesh=plsc.VectorSubcoreMesh(
        core_axis_name="c", subcore_axis_name="s",
        num_cores=_NC, num_subcores=_NS),
    compiler_params=pltpu.CompilerParams(use_tc_tiling_on_sc=False),
    scratch_types=[
        pltpu.VMEM((_EXT, _D), jnp.float32),
        pltpu.VMEM((_BPW,), jnp.int32),
        pltpu.VMEM((_LANES,), jnp.float32),
        pltpu.VMEM((_LANES,), jnp.int32),
        pltpu.VMEM((_BPW,), jnp.int32),
    ],
)
def _cycle_gather(phase_hbm, w_hbm, off_hbm, base_hbm, out_hbm,
                  table_v, phase_v, w_v, off_v, starts_v):
    wid = lax.axis_index("s") * _NC + lax.axis_index("c")
    bbase = wid * _BPW

    pltpu.sync_copy(base_hbm, table_v.at[pl.ds(0, _CYCLE)])
    pltpu.sync_copy(base_hbm.at[pl.ds(0, _L - 1)],
                    table_v.at[pl.ds(_CYCLE, _L - 1)])
    pltpu.sync_copy(phase_hbm.at[pl.ds(bbase, _BPW)], phase_v)
    pltpu.sync_copy(w_hbm, w_v)
    pltpu.sync_copy(off_hbm, off_v)

    w = w_v[...]
    off = off_v[...]
    for i in range(_BPW // _LANES):
        ph = phase_v[pl.ds(i * _LANES, _LANES)]
        shifted = (ph.astype(jnp.float32) * w).astype(jnp.int32) + off
        r = lax.rem(shifted, _CYCLE)
        r = jnp.where(r < 0, r + _CYCLE, r)
        starts_v[pl.ds(i * _LANES, _LANES)] = r

    for g in range(_BPW // _LANES):
        sv = starts_v[pl.ds(g * _LANES, _LANES)]
        for j in range(_LANES):
            s = sv[j]
            pltpu.sync_copy(table_v.at[pl.ds(s, _L)],
                            out_hbm.at[bbase + g * _LANES + j])


def kernel(phase, length, base, w_ps):
    w16 = jnp.broadcast_to(jnp.reshape(w_ps, (1,)).astype(jnp.float32),
                           (_LANES,))
    off16 = jnp.broadcast_to(
        jnp.reshape(jnp.asarray(length, jnp.int32) - _L, (1,)), (_LANES,))
    return _cycle_gather(phase, w16, off16, base)


# speedup vs baseline: 5.1584x; 1.0021x over previous
import functools

import jax
import jax.numpy as jnp
from jax import lax
from jax.experimental import pallas as pl
from jax.experimental.pallas import tpu as pltpu
from jax.experimental.pallas import tpu_sc as plsc

_CYCLE = 1000
_L = 200
_D = 64
_B = 4096
_EXT = _CYCLE + _L - 1

_NC = 2
_NS = 16
_NW = _NC * _NS
_BPW = _B // _NW
_LANES = 16


@functools.partial(
    pl.kernel,
    out_type=jax.ShapeDtypeStruct((_B, _L, _D), jnp.float32),
    mesh=plsc.VectorSubcoreMesh(
        core_axis_name="c", subcore_axis_name="s",
        num_cores=_NC, num_subcores=_NS),
    compiler_params=pltpu.CompilerParams(use_tc_tiling_on_sc=False),
    scratch_types=[
        pltpu.VMEM((_EXT, _D), jnp.float32),
        pltpu.VMEM((_BPW,), jnp.int32),
        pltpu.VMEM((_LANES,), jnp.float32),
        pltpu.VMEM((_LANES,), jnp.int32),
        pltpu.VMEM((_BPW,), jnp.int32),
        pltpu.SemaphoreType.DMA,
    ],
)
def _cycle_gather(phase_hbm, w_hbm, off_hbm, base_hbm, out_hbm,
                  table_v, phase_v, w_v, off_v, starts_v, sem):
    wid = lax.axis_index("s") * _NC + lax.axis_index("c")
    bbase = wid * _BPW

    pltpu.sync_copy(base_hbm, table_v.at[pl.ds(0, _CYCLE)])
    pltpu.sync_copy(base_hbm.at[pl.ds(0, _L - 1)],
                    table_v.at[pl.ds(_CYCLE, _L - 1)])
    pltpu.sync_copy(phase_hbm.at[pl.ds(bbase, _BPW)], phase_v)
    pltpu.sync_copy(w_hbm, w_v)
    pltpu.sync_copy(off_hbm, off_v)

    w = w_v[...]
    off = off_v[...]
    for i in range(_BPW // _LANES):
        ph = phase_v[pl.ds(i * _LANES, _LANES)]
        shifted = (ph.astype(jnp.float32) * w).astype(jnp.int32) + off
        r = lax.rem(shifted, _CYCLE)
        r = jnp.where(r < 0, r + _CYCLE, r)
        starts_v[pl.ds(i * _LANES, _LANES)] = r

    window = 32
    pending = []
    for g in range(_BPW // _LANES):
        sv = starts_v[pl.ds(g * _LANES, _LANES)]
        for j in range(_LANES):
            s = sv[j]
            c = pltpu.async_copy(table_v.at[pl.ds(s, _L)],
                                 out_hbm.at[bbase + g * _LANES + j], sem)
            pending.append(c)
            if len(pending) > window:
                pending.pop(0).wait()
    for c in pending:
        c.wait()


def kernel(phase, length, base, w_ps):
    w16 = jnp.broadcast_to(jnp.reshape(w_ps, (1,)).astype(jnp.float32),
                           (_LANES,))
    off16 = jnp.broadcast_to(
        jnp.reshape(jnp.asarray(length, jnp.int32) - _L, (1,)), (_LANES,))
    return _cycle_gather(phase, w16, off16, base)


# speedup vs baseline: 6.8247x; 1.3230x over previous
import functools

import jax
import jax.numpy as jnp
from jax import lax
from jax.experimental import pallas as pl
from jax.experimental.pallas import tpu as pltpu
from jax.experimental.pallas import tpu_sc as plsc

_CYCLE = 1000
_L = 200
_D = 64
_B = 4096
_EXT = _CYCLE + _L - 1

_NC = 2
_NS = 16
_NW = _NC * _NS
_BPW = _B // _NW
_LANES = 16


@functools.partial(
    pl.kernel,
    out_type=jax.ShapeDtypeStruct((_B, _L * _D), jnp.float32),
    mesh=plsc.VectorSubcoreMesh(
        core_axis_name="c", subcore_axis_name="s",
        num_cores=_NC, num_subcores=_NS),
    compiler_params=pltpu.CompilerParams(use_tc_tiling_on_sc=False),
    scratch_types=[
        pltpu.VMEM((_EXT * _D,), jnp.float32),
        pltpu.VMEM((_BPW,), jnp.int32),
        pltpu.VMEM((_LANES,), jnp.float32),
        pltpu.VMEM((_LANES,), jnp.int32),
        pltpu.VMEM((_BPW,), jnp.int32),
        pltpu.SemaphoreType.DMA,
    ],
)
def _cycle_gather(phase_hbm, w_hbm, off_hbm, base_hbm, out_hbm,
                  table_v, phase_v, w_v, off_v, starts_v, sem):
    wid = lax.axis_index("s") * _NC + lax.axis_index("c")
    bbase = wid * _BPW

    pltpu.sync_copy(base_hbm, table_v.at[pl.ds(0, _CYCLE * _D)])
    pltpu.sync_copy(base_hbm.at[pl.ds(0, (_L - 1) * _D)],
                    table_v.at[pl.ds(_CYCLE * _D, (_L - 1) * _D)])
    pltpu.sync_copy(phase_hbm.at[pl.ds(bbase, _BPW)], phase_v)
    pltpu.sync_copy(w_hbm, w_v)
    pltpu.sync_copy(off_hbm, off_v)

    w = w_v[...]
    off = off_v[...]
    for i in range(_BPW // _LANES):
        ph = phase_v[pl.ds(i * _LANES, _LANES)]
        shifted = (ph.astype(jnp.float32) * w).astype(jnp.int32) + off
        r = lax.rem(shifted, _CYCLE)
        r = jnp.where(r < 0, r + _CYCLE, r)
        starts_v[pl.ds(i * _LANES, _LANES)] = r

    window = 32
    pending = []
    for g in range(_BPW // _LANES):
        sv = starts_v[pl.ds(g * _LANES, _LANES)]
        for j in range(_LANES):
            s = sv[j]
            c = pltpu.async_copy(table_v.at[pl.ds(s * _D, _L * _D)],
                                 out_hbm.at[bbase + g * _LANES + j], sem)
            pending.append(c)
            if len(pending) > window:
                pending.pop(0).wait()
    for c in pending:
        c.wait()


def kernel(phase, length, base, w_ps):
    w16 = jnp.broadcast_to(jnp.reshape(w_ps, (1,)).astype(jnp.float32),
                           (_LANES,))
    off16 = jnp.broadcast_to(
        jnp.reshape(jnp.asarray(length, jnp.int32) - _L, (1,)), (_LANES,))
    flat = _cycle_gather(phase, w16, off16, jnp.reshape(base, (-1,)))
    return jnp.reshape(flat, (_B, _L, _D))
